# baseline (speedup 1.0000x reference)
import functools

import jax
import jax.numpy as jnp
from jax.experimental import pallas as pl
from jax.experimental.pallas import tpu as pltpu

_N_ROWS = 10000
_N_COLS = 128
_N_CHUNKS = 2
_CHUNK_ROWS = _N_ROWS // _N_CHUNKS


def _body(x_hbm, h_hbm, u_hbm, t_hbm, buf, ut_buf, in_sems, out_sems, ut_sems):
    def in_copy(c):
        return pltpu.make_async_copy(
            x_hbm.at[pl.ds(c * _CHUNK_ROWS, _CHUNK_ROWS), :],
            buf.at[c],
            in_sems.at[c],
        )

    def out_copy(c):
        return pltpu.make_async_copy(
            buf.at[c],
            h_hbm.at[pl.ds(c * _CHUNK_ROWS, _CHUNK_ROWS), :],
            out_sems.at[c],
        )

    for c in range(_N_CHUNKS):
        in_copy(c).start()

    acc = jnp.zeros((1, _N_COLS), jnp.float32)
    for c in range(_N_CHUNKS):
        in_copy(c).wait()
        out_copy(c).start()
        acc = acc + jnp.sum(buf[c], axis=0, keepdims=True)

    u = acc * (1.0 / _N_ROWS)
    ut_buf[0:1, :] = u
    m = jnp.max(u, axis=1, keepdims=True)
    e = jnp.exp(u - m)
    ut_buf[1:2, :] = e / jnp.sum(e, axis=1, keepdims=True)
    pltpu.make_async_copy(ut_buf.at[0:1, :], u_hbm, ut_sems.at[0]).start()
    pltpu.make_async_copy(ut_buf.at[1:2, :], t_hbm, ut_sems.at[1]).start()

    pltpu.make_async_copy(ut_buf.at[0:1, :], u_hbm, ut_sems.at[0]).wait()
    pltpu.make_async_copy(ut_buf.at[1:2, :], t_hbm, ut_sems.at[1]).wait()
    for c in range(_N_CHUNKS):
        out_copy(c).wait()


@functools.partial(jax.jit, static_argnames=())
def _fused(x):
    h, u, u_top = pl.pallas_call(
        _body,
        in_specs=[pl.BlockSpec(memory_space=pltpu.MemorySpace.HBM)],
        out_specs=[
            pl.BlockSpec(memory_space=pltpu.MemorySpace.HBM),
            pl.BlockSpec(memory_space=pltpu.MemorySpace.HBM),
            pl.BlockSpec(memory_space=pltpu.MemorySpace.HBM),
        ],
        out_shape=[
            jax.ShapeDtypeStruct((_N_ROWS, _N_COLS), jnp.float32),
            jax.ShapeDtypeStruct((1, _N_COLS), jnp.float32),
            jax.ShapeDtypeStruct((1, _N_COLS), jnp.float32),
        ],
        scratch_shapes=[
            pltpu.VMEM((_N_CHUNKS, _CHUNK_ROWS, _N_COLS), jnp.float32),
            pltpu.VMEM((2, _N_COLS), jnp.float32),
            pltpu.SemaphoreType.DMA((_N_CHUNKS,)),
            pltpu.SemaphoreType.DMA((_N_CHUNKS,)),
            pltpu.SemaphoreType.DMA((2,)),
        ],
    )(x)
    return h, u, u_top


def kernel(x, edge_index):
    del edge_index
    return _fused(x)


# speedup vs baseline: 1.0041x; 1.0041x over previous
import functools

import jax
import jax.numpy as jnp
from jax.experimental import pallas as pl
from jax.experimental.pallas import tpu as pltpu

_N_ROWS = 10000
_N_COLS = 128
_N_CHUNKS = 5
_CHUNK_ROWS = _N_ROWS // _N_CHUNKS


def _body(x_hbm, h_hbm, u_hbm, t_hbm, buf, ut_buf, in_sems, out_sems, ut_sems):
    def in_copy(c):
        return pltpu.make_async_copy(
            x_hbm.at[pl.ds(c * _CHUNK_ROWS, _CHUNK_ROWS), :],
            buf.at[c],
            in_sems.at[c],
        )

    def out_copy(c):
        return pltpu.make_async_copy(
            buf.at[c],
            h_hbm.at[pl.ds(c * _CHUNK_ROWS, _CHUNK_ROWS), :],
            out_sems.at[c],
        )

    for c in range(_N_CHUNKS):
        in_copy(c).start()

    acc = jnp.zeros((1, _N_COLS), jnp.float32)
    for c in range(_N_CHUNKS):
        in_copy(c).wait()
        out_copy(c).start()
        acc = acc + jnp.sum(buf[c], axis=0, keepdims=True)

    u = acc * (1.0 / _N_ROWS)
    ut_buf[0:1, :] = u
    m = jnp.max(u, axis=1, keepdims=True)
    e = jnp.exp(u - m)
    ut_buf[1:2, :] = e / jnp.sum(e, axis=1, keepdims=True)
    pltpu.make_async_copy(ut_buf.at[0:1, :], u_hbm, ut_sems.at[0]).start()
    pltpu.make_async_copy(ut_buf.at[1:2, :], t_hbm, ut_sems.at[1]).start()

    pltpu.make_async_copy(ut_buf.at[0:1, :], u_hbm, ut_sems.at[0]).wait()
    pltpu.make_async_copy(ut_buf.at[1:2, :], t_hbm, ut_sems.at[1]).wait()
    for c in range(_N_CHUNKS):
        out_copy(c).wait()


@functools.partial(jax.jit, static_argnames=())
def _fused(x):
    h, u, u_top = pl.pallas_call(
        _body,
        in_specs=[pl.BlockSpec(memory_space=pltpu.MemorySpace.HBM)],
        out_specs=[
            pl.BlockSpec(memory_space=pltpu.MemorySpace.HBM),
            pl.BlockSpec(memory_space=pltpu.MemorySpace.HBM),
            pl.BlockSpec(memory_space=pltpu.MemorySpace.HBM),
        ],
        out_shape=[
            jax.ShapeDtypeStruct((_N_ROWS, _N_COLS), jnp.float32),
            jax.ShapeDtypeStruct((1, _N_COLS), jnp.float32),
            jax.ShapeDtypeStruct((1, _N_COLS), jnp.float32),
        ],
        scratch_shapes=[
            pltpu.VMEM((_N_CHUNKS, _CHUNK_ROWS, _N_COLS), jnp.float32),
            pltpu.VMEM((2, _N_COLS), jnp.float32),
            pltpu.SemaphoreType.DMA((_N_CHUNKS,)),
            pltpu.SemaphoreType.DMA((_N_CHUNKS,)),
            pltpu.SemaphoreType.DMA((2,)),
        ],
    )(x)
    return h, u, u_top


def kernel(x, edge_index):
    del edge_index
    return _fused(x)


# speedup vs baseline: 1.0163x; 1.0121x over previous
import functools

import jax
import jax.numpy as jnp
from jax.experimental import pallas as pl
from jax.experimental.pallas import tpu as pltpu

_N_ROWS = 10000
_N_COLS = 128
_N_CHUNKS = 4
_CHUNK_ROWS = _N_ROWS // _N_CHUNKS


def _body(x_hbm, h_hbm, u_hbm, t_hbm, buf, ut_buf, in_sems, out_sems, ut_sems):
    def in_copy(c):
        return pltpu.make_async_copy(
            x_hbm.at[pl.ds(c * _CHUNK_ROWS, _CHUNK_ROWS), :],
            buf.at[c],
            in_sems.at[c],
        )

    def out_copy(c):
        return pltpu.make_async_copy(
            buf.at[c],
            h_hbm.at[pl.ds(c * _CHUNK_ROWS, _CHUNK_ROWS), :],
            out_sems.at[c],
        )

    for c in range(_N_CHUNKS):
        in_copy(c).start()

    acc = jnp.zeros((1, _N_COLS), jnp.float32)
    for c in range(_N_CHUNKS):
        in_copy(c).wait()
        out_copy(c).start()
        acc = acc + jnp.sum(buf[c], axis=0, keepdims=True)

    u = acc * (1.0 / _N_ROWS)
    ut_buf[0:1, :] = u
    m = jnp.max(u, axis=1, keepdims=True)
    e = jnp.exp(u - m)
    ut_buf[1:2, :] = e / jnp.sum(e, axis=1, keepdims=True)
    pltpu.make_async_copy(ut_buf.at[0:1, :], u_hbm, ut_sems.at[0]).start()
    pltpu.make_async_copy(ut_buf.at[1:2, :], t_hbm, ut_sems.at[1]).start()

    pltpu.make_async_copy(ut_buf.at[0:1, :], u_hbm, ut_sems.at[0]).wait()
    pltpu.make_async_copy(ut_buf.at[1:2, :], t_hbm, ut_sems.at[1]).wait()
    for c in range(_N_CHUNKS):
        out_copy(c).wait()


@functools.partial(jax.jit, static_argnames=())
def _fused(x):
    h, u, u_top = pl.pallas_call(
        _body,
        in_specs=[pl.BlockSpec(memory_space=pltpu.MemorySpace.HBM)],
        out_specs=[
            pl.BlockSpec(memory_space=pltpu.MemorySpace.HBM),
            pl.BlockSpec(memory_space=pltpu.MemorySpace.HBM),
            pl.BlockSpec(memory_space=pltpu.MemorySpace.HBM),
        ],
        out_shape=[
            jax.ShapeDtypeStruct((_N_ROWS, _N_COLS), jnp.float32),
            jax.ShapeDtypeStruct((1, _N_COLS), jnp.float32),
            jax.ShapeDtypeStruct((1, _N_COLS), jnp.float32),
        ],
        scratch_shapes=[
            pltpu.VMEM((_N_CHUNKS, _CHUNK_ROWS, _N_COLS), jnp.float32),
            pltpu.VMEM((2, _N_COLS), jnp.float32),
            pltpu.SemaphoreType.DMA((_N_CHUNKS,)),
            pltpu.SemaphoreType.DMA((_N_CHUNKS,)),
            pltpu.SemaphoreType.DMA((2,)),
        ],
    )(x)
    return h, u, u_top


def kernel(x, edge_index):
    del edge_index
    return _fused(x)
